# baseline (speedup 1.0000x reference)
import functools

import numpy as np

import jax
import jax.numpy as jnp
from jax import lax
from jax.experimental import pallas as pl
from jax.experimental.pallas import tpu as pltpu
from jax.experimental.pallas import tpu_sc as plsc

N = 10000
E = 320000
F = 128
H = 16

NC, NS = 2, 16
NW = NC * NS
EPW = E // NW
CHUNK = 80
NCHUNK = EPW // CHUNK
RPW = 632
NPAD = NS * RPW
PN = N // 8
PP = NPAD // 8
PE = E // 8

_mesh = plsc.VectorSubcoreMesh(core_axis_name="c", subcore_axis_name="s")
_sc_params = pltpu.CompilerParams(use_tc_tiling_on_sc=False)


def _zero_shared(acc_sh, zbuf, sid):
    zrow = jnp.zeros((16,), jnp.float32)

    @pl.loop(0, RPW)
    def _(i):
        zbuf[i] = zrow

    pltpu.sync_copy(zbuf, acc_sh.at[pl.ds(sid * RPW, RPW)])


def _drain_shared(acc_sh, zbuf, out_hbm, core, sid):
    sl = pl.ds(sid * RPW, RPW)
    pltpu.sync_copy(acc_sh.at[sl], zbuf)
    pltpu.sync_copy(zbuf, out_hbm.at[core, sl])


def _sc_degree(dst3):
    @functools.partial(
        pl.kernel,
        out_type=jax.ShapeDtypeStruct((NC, NPAD, 16), jnp.float32),
        mesh=_mesh,
        compiler_params=_sc_params,
        scratch_types=[
            pltpu.VMEM((RPW, 16), jnp.float32),
            pltpu.VMEM((NCHUNK, CHUNK), jnp.int32),
            pltpu.VMEM((CHUNK, 16), jnp.float32),
            pltpu.VMEM_SHARED((NPAD, 16), jnp.float32),
        ],
    )
    def k(dst_hbm, out_hbm, zbuf, didx, ones_v, acc_sh):
        core = lax.axis_index("c")
        sid = lax.axis_index("s")
        wid = core * NS + sid

        _zero_shared(acc_sh, zbuf, sid)

        one = jnp.ones((16,), jnp.float32)

        @pl.loop(0, CHUNK)
        def _(i):
            ones_v[i] = one

        pltpu.sync_copy(dst_hbm.at[wid], didx)
        plsc.subcore_barrier()

        @pl.loop(0, NCHUNK)
        def _(j):
            pltpu.sync_copy(ones_v, acc_sh.at[didx.at[j]], add=True)

        plsc.subcore_barrier()
        _drain_shared(acc_sh, zbuf, out_hbm, core, sid)

    return k(dst3)


NBUF = 5
NROUND = NCHUNK // NBUF


def _sc_conv(hp, src3, dst3):
    @functools.partial(
        pl.kernel,
        out_type=jax.ShapeDtypeStruct((NC, NPAD, 16), jnp.float32),
        mesh=_mesh,
        compiler_params=_sc_params,
        scratch_types=[
            pltpu.VMEM((RPW, 16), jnp.float32),
            pltpu.VMEM((NCHUNK, CHUNK), jnp.int32),
            pltpu.VMEM((NCHUNK, CHUNK), jnp.int32),
            pltpu.VMEM((NBUF, CHUNK, 16), jnp.float32),
            pltpu.VMEM_SHARED((NPAD, 16), jnp.float32),
            pltpu.SemaphoreType.DMA((NBUF,)),
        ],
    )
    def k(hp_hbm, src_hbm, dst_hbm, out_hbm, zbuf, sidx, didx, rows, acc_sh,
          gsem):
        core = lax.axis_index("c")
        sid = lax.axis_index("s")
        wid = core * NS + sid

        _zero_shared(acc_sh, zbuf, sid)
        pltpu.sync_copy(src_hbm.at[wid], sidx)
        pltpu.sync_copy(dst_hbm.at[wid], didx)
        plsc.subcore_barrier()

        def issue(b, jj):
            pltpu.async_copy(hp_hbm.at[sidx.at[jj]], rows.at[b], gsem.at[b])

        def wait(b):
            pltpu.make_async_copy(hp_hbm.at[sidx.at[0]], rows.at[b],
                                  gsem.at[b]).wait()

        for b in range(NBUF):
            issue(b, b)

        @pl.loop(0, NROUND - 1)
        def _(r):
            for b in range(NBUF):
                jj = r * NBUF + b
                wait(b)
                pltpu.sync_copy(rows.at[b], acc_sh.at[didx.at[jj]], add=True)
                issue(b, jj + NBUF)

        for b in range(NBUF):
            jj = (NROUND - 1) * NBUF + b
            wait(b)
            pltpu.sync_copy(rows.at[b], acc_sh.at[didx.at[jj]], add=True)

        plsc.subcore_barrier()
        _drain_shared(acc_sh, zbuf, out_hbm, core, sid)

    return k(hp, src3, dst3)


def _sc_edge_pre(A3, B3, src3, dst3):
    @functools.partial(
        pl.kernel,
        out_type=jax.ShapeDtypeStruct((E, 16), jnp.float32),
        mesh=_mesh,
        compiler_params=_sc_params,
        scratch_types=[
            pltpu.VMEM((NCHUNK, CHUNK), jnp.int32),
            pltpu.VMEM((NCHUNK, CHUNK), jnp.int32),
            pltpu.VMEM((NBUF, CHUNK, 16), jnp.float32),
            pltpu.VMEM((NBUF, CHUNK, 16), jnp.float32),
            pltpu.VMEM((NBUF, CHUNK, 16), jnp.float32),
            pltpu.SemaphoreType.DMA((NBUF,)),
            pltpu.SemaphoreType.DMA((NBUF,)),
            pltpu.SemaphoreType.DMA((NBUF,)),
        ],
    )
    def k(a_hbm, b_hbm, src_hbm, dst_hbm, out_hbm, sidx, didx, ga, gb, wo,
          gsa, gsb, wsem):
        core = lax.axis_index("c")
        sid = lax.axis_index("s")
        wid = core * NS + sid
        base = wid * EPW

        pltpu.sync_copy(src_hbm.at[wid], sidx)
        pltpu.sync_copy(dst_hbm.at[wid], didx)

        def issue(b, jj):
            pltpu.async_copy(a_hbm.at[sidx.at[jj]], ga.at[b], gsa.at[b])
            pltpu.async_copy(b_hbm.at[didx.at[jj]], gb.at[b], gsb.at[b])

        def out_slice(jj):
            return out_hbm.at[pl.ds(base + jj * CHUNK, CHUNK)]

        def process(jj, b, first):
            pltpu.make_async_copy(a_hbm.at[sidx.at[0]], ga.at[b],
                                  gsa.at[b]).wait()
            pltpu.make_async_copy(b_hbm.at[didx.at[0]], gb.at[b],
                                  gsb.at[b]).wait()
            if not first:
                pltpu.make_async_copy(wo.at[b], out_slice(jj),
                                      wsem.at[b]).wait()

            @pl.loop(0, CHUNK)
            def _(c):
                wo.at[b][c] = ga.at[b][c] + gb.at[b][c]

            pltpu.async_copy(wo.at[b], out_slice(jj), wsem.at[b])

        for b in range(NBUF):
            issue(b, b)
        for b in range(NBUF):
            process(b, b, True)
            issue(b, b + NBUF)

        @pl.loop(1, NROUND - 1)
        def _(r):
            for b in range(NBUF):
                jj = r * NBUF + b
                process(jj, b, False)
                issue(b, jj + NBUF)

        for b in range(NBUF):
            jj = (NROUND - 1) * NBUF + b
            process(jj, b, False)
        for b in range(NBUF):
            pltpu.make_async_copy(wo.at[b], out_slice(0), wsem.at[b]).wait()

    return k(A3, B3, src3, dst3)


def _tc_pre(deg_parts, x, W1, b1):
    def body(dp_ref, x_ref, w_ref, dinv_ref, hp_ref):
        indeg = dp_ref[0, :PN, :] + dp_ref[1, :PN, :]
        dinv = lax.rsqrt(indeg + 1.0)
        dinv_ref[...] = dinv
        hw = jnp.dot(x_ref[...], w_ref[...], preferred_element_type=jnp.float32)
        hp_ref[...] = dinv * hw

    return pl.pallas_call(
        body,
        out_shape=(
            jax.ShapeDtypeStruct((PN, 128), jnp.float32),
            jax.ShapeDtypeStruct((PN, 128), jnp.float32),
        ),
    )(deg_parts, x, W1)


def _tc_post(parts, hp, dinv, b128, Wbd, res=None):
    args = [parts, hp, dinv, b128, Wbd] + ([res] if res is not None else [])

    def body(p_ref, hp_ref, dinv_ref, b_ref, w_ref, *rest):
        (res_ref, h_ref, hpn_ref) = rest if len(rest) == 3 else \
            (None,) + rest
        acc = p_ref[0, :PN, :] + p_ref[1, :PN, :] + hp_ref[...]
        out = dinv_ref[...] * acc + b_ref[...]
        h = jnp.maximum(out, 0.0)
        if res_ref is not None:
            h = h + res_ref[...]
        h_ref[...] = h
        hw = jnp.dot(h, w_ref[...], preferred_element_type=jnp.float32)
        hpn_ref[...] = dinv_ref[...] * hw

    return pl.pallas_call(
        body,
        out_shape=(
            jax.ShapeDtypeStruct((PN, 128), jnp.float32),
            jax.ShapeDtypeStruct((PN, 128), jnp.float32),
        ),
    )(*args)


def _tc_post3(parts, hp, dinv, b128, Wabd, Wbbd, bf1_128):
    def body(p_ref, hp_ref, dinv_ref, b_ref, wa_ref, wb_ref, bf1_ref,
             a_ref, bo_ref):
        acc = p_ref[0, :PN, :] + p_ref[1, :PN, :] + hp_ref[...]
        h3 = dinv_ref[...] * acc + b_ref[...]
        a_ref[...] = jnp.dot(h3, wa_ref[...],
                             preferred_element_type=jnp.float32) + bf1_ref[...]
        bo_ref[...] = jnp.dot(h3, wb_ref[...],
                              preferred_element_type=jnp.float32)

    return pl.pallas_call(
        body,
        out_shape=(
            jax.ShapeDtypeStruct((PN, 128), jnp.float32),
            jax.ShapeDtypeStruct((PN, 128), jnp.float32),
        ),
    )(parts, hp, dinv, b128, Wabd, Wbbd, bf1_128)


EDGE_BLOCK = 4000


def _group_perm(k):
    j = np.arange(128)
    src = (j // 16) * 16 + ((j % 16 + k) % 16)
    p = np.zeros((128, 128), np.float32)
    p[src, j] = 1.0
    return jnp.asarray(p)


def _tc_final(pre_p, Wf2bd, bf2_128, bdones, perms):
    def body(pre_ref, w_ref, b_ref, ones_ref, p1, p2, p4, p8, out_ref):
        ef = jnp.maximum(pre_ref[...], 0.0)
        logits = jnp.dot(ef, w_ref[...], preferred_element_type=jnp.float32)
        logits = logits + b_ref[...]
        m = logits
        for p_ref in (p1, p2, p4, p8):
            m = jnp.maximum(m, jnp.dot(m, p_ref[...],
                                       preferred_element_type=jnp.float32))
        s = logits - m
        se = jnp.dot(jnp.exp(s), ones_ref[...],
                     preferred_element_type=jnp.float32)
        out_ref[...] = s - jnp.log(se)

    full = lambda i: (0, 0)
    return pl.pallas_call(
        body,
        grid=(PE // EDGE_BLOCK,),
        in_specs=[pl.BlockSpec((EDGE_BLOCK, 128), lambda i: (i, 0))] +
                 [pl.BlockSpec((128, 128), full)] +
                 [pl.BlockSpec((1, 128), full)] +
                 [pl.BlockSpec((128, 128), full)] * 5,
        out_specs=pl.BlockSpec((EDGE_BLOCK, 128), lambda i: (i, 0)),
        out_shape=jax.ShapeDtypeStruct((PE, 128), jnp.float32),
    )(pre_p, Wf2bd, bf2_128, bdones, *perms)


OPW = E // NW
OCH = 1000
OSLOTS = 4


def _sc_copy_out(src_arr):
    nch = OPW // OCH

    @functools.partial(
        pl.kernel,
        out_type=jax.ShapeDtypeStruct((E, 16), jnp.float32),
        mesh=_mesh,
        compiler_params=_sc_params,
        scratch_types=[
            pltpu.VMEM((OSLOTS, OCH, 16), jnp.float32),
            pltpu.SemaphoreType.DMA((OSLOTS,)),
            pltpu.SemaphoreType.DMA((OSLOTS,)),
        ],
    )
    def k(in_hbm, out_hbm, buf, isem, osem):
        core = lax.axis_index("c")
        sid = lax.axis_index("s")
        wid = core * NS + sid
        base = wid * OPW

        def rd(b, j):
            sl = pl.ds(base + j * OCH, OCH)
            pltpu.async_copy(in_hbm.at[sl], buf.at[b], isem.at[b])

        def rd_wait(b):
            pltpu.make_async_copy(in_hbm.at[pl.ds(0, OCH)], buf.at[b],
                                  isem.at[b]).wait()

        def wr(b, j):
            sl = pl.ds(base + j * OCH, OCH)
            pltpu.async_copy(buf.at[b], out_hbm.at[sl], osem.at[b])

        def wr_wait(b):
            pltpu.make_async_copy(buf.at[b], out_hbm.at[pl.ds(0, OCH)],
                                  osem.at[b]).wait()

        for j in range(nch + 2):
            if j < nch:
                b = j % OSLOTS
                if j >= OSLOTS:
                    wr_wait(b)
                rd(b, j)
            if j >= 2:
                i = j - 2
                b = i % OSLOTS
                rd_wait(b)
                wr(b, i)
        for b in range(OSLOTS):
            wr_wait(b)

    return k(src_arr)


def _bd(W):
    return jnp.kron(jnp.eye(8, dtype=W.dtype), W)


def kernel(x, edge_index, W1, b1, W2, b2, W3, b3, Wf1, bf1, Wf2, bf2):
    src3 = edge_index[0].astype(jnp.int32).reshape(NW, NCHUNK, CHUNK)
    dst3 = edge_index[1].astype(jnp.int32).reshape(NW, NCHUNK, CHUNK)

    x_r = x.reshape(PN, 8 * F)
    W1bd = _bd(W1)
    W2bd, W3bd = _bd(W2), _bd(W3)
    Wabd, Wbbd = _bd(Wf1[:16]), _bd(Wf1[16:])
    Wf2bd = _bd(Wf2)
    t8 = lambda b: jnp.tile(b, 8).reshape(1, 128)

    deg_parts = _sc_degree(dst3).reshape(NC, PP, 128)
    dinv, hp1 = _tc_pre(deg_parts, x_r, W1bd, b1)

    p1 = _sc_conv(hp1.reshape(N, 16), src3, dst3).reshape(NC, PP, 128)
    h1, hp2 = _tc_post(p1, hp1, dinv, t8(b1), W2bd)

    p2 = _sc_conv(hp2.reshape(N, 16), src3, dst3).reshape(NC, PP, 128)
    h2, hp3 = _tc_post(p2, hp2, dinv, t8(b2), W3bd, res=h1)

    p3 = _sc_conv(hp3.reshape(N, 16), src3, dst3).reshape(NC, PP, 128)
    A3, B3 = _tc_post3(p3, hp3, dinv, t8(b3), Wabd, Wbbd, t8(bf1))

    pre = _sc_edge_pre(A3.reshape(N, 16), B3.reshape(N, 16), src3, dst3)
    bdones = jnp.asarray(np.kron(np.eye(8, dtype=np.float32),
                                  np.ones((16, 16), np.float32)))
    perms = [_group_perm(k) for k in (1, 2, 4, 8)]
    out_p = _tc_final(pre.reshape(PE, 128), Wf2bd, t8(bf2), bdones, perms)
    return _sc_copy_out(out_p.reshape(E, 16))


# speedup vs baseline: 1.0613x; 1.0613x over previous
import functools

import numpy as np

import jax
import jax.numpy as jnp
from jax import lax
from jax.experimental import pallas as pl
from jax.experimental.pallas import tpu as pltpu
from jax.experimental.pallas import tpu_sc as plsc

N = 10000
E = 320000
F = 128
H = 16

NC, NS = 2, 16
NW = NC * NS
EPW = E // NW
CHUNK = 80
NCHUNK = EPW // CHUNK
RPW = 632
NPAD = NS * RPW
PN = N // 8
PP = NPAD // 8
PE = E // 8

_mesh = plsc.VectorSubcoreMesh(core_axis_name="c", subcore_axis_name="s")
_sc_params = pltpu.CompilerParams(use_tc_tiling_on_sc=False)


def _zero_shared(acc_sh, zbuf, sid):
    zrow = jnp.zeros((16,), jnp.float32)

    @pl.loop(0, RPW)
    def _(i):
        zbuf[i] = zrow

    pltpu.sync_copy(zbuf, acc_sh.at[pl.ds(sid * RPW, RPW)])


def _drain_shared(acc_sh, zbuf, out_hbm, core, sid):
    sl = pl.ds(sid * RPW, RPW)
    pltpu.sync_copy(acc_sh.at[sl], zbuf)
    pltpu.sync_copy(zbuf, out_hbm.at[core, sl])


def _sc_degree(dst3):
    @functools.partial(
        pl.kernel,
        out_type=jax.ShapeDtypeStruct((NC, NPAD, 16), jnp.float32),
        mesh=_mesh,
        compiler_params=_sc_params,
        scratch_types=[
            pltpu.VMEM((RPW, 16), jnp.float32),
            pltpu.VMEM((NCHUNK, CHUNK), jnp.int32),
            pltpu.VMEM((CHUNK, 16), jnp.float32),
            pltpu.VMEM_SHARED((NPAD, 16), jnp.float32),
        ],
    )
    def k(dst_hbm, out_hbm, zbuf, didx, ones_v, acc_sh):
        core = lax.axis_index("c")
        sid = lax.axis_index("s")
        wid = core * NS + sid

        _zero_shared(acc_sh, zbuf, sid)

        one = jnp.ones((16,), jnp.float32)

        @pl.loop(0, CHUNK)
        def _(i):
            ones_v[i] = one

        pltpu.sync_copy(dst_hbm.at[wid], didx)
        plsc.subcore_barrier()

        @pl.loop(0, NCHUNK)
        def _(j):
            pltpu.sync_copy(ones_v, acc_sh.at[didx.at[j]], add=True)

        plsc.subcore_barrier()
        _drain_shared(acc_sh, zbuf, out_hbm, core, sid)

    return k(dst3)


NBUF = 5
NROUND = NCHUNK // NBUF


def _sc_conv(hp, src3, dst3):
    NB2 = 2 * NBUF

    @functools.partial(
        pl.kernel,
        out_type=jax.ShapeDtypeStruct((NC, NPAD, 16), jnp.float32),
        mesh=_mesh,
        compiler_params=_sc_params,
        scratch_types=[
            pltpu.VMEM((RPW, 16), jnp.float32),
            pltpu.VMEM((NCHUNK, CHUNK), jnp.int32),
            pltpu.VMEM((NCHUNK, CHUNK), jnp.int32),
            pltpu.VMEM((2 * NBUF, CHUNK, 16), jnp.float32),
            pltpu.VMEM_SHARED((NPAD, 16), jnp.float32),
            pltpu.SemaphoreType.DMA((2 * NBUF,)),
            pltpu.SemaphoreType.DMA((2 * NBUF,)),
        ],
    )
    def k(hp_hbm, src_hbm, dst_hbm, out_hbm, zbuf, sidx, didx, rows, acc_sh,
          gsem, ssem):
        core = lax.axis_index("c")
        sid = lax.axis_index("s")
        wid = core * NS + sid

        _zero_shared(acc_sh, zbuf, sid)
        pltpu.sync_copy(src_hbm.at[wid], sidx)
        pltpu.sync_copy(dst_hbm.at[wid], didx)
        plsc.subcore_barrier()

        def g_issue(b, jj):
            pltpu.async_copy(hp_hbm.at[sidx.at[jj]], rows.at[b], gsem.at[b])

        def g_wait(b):
            pltpu.make_async_copy(hp_hbm.at[sidx.at[0]], rows.at[b],
                                  gsem.at[b]).wait()

        def s_issue(b, jj):
            pltpu.async_copy(rows.at[b], acc_sh.at[didx.at[jj]], ssem.at[b],
                             add=True)

        def s_wait(b):
            pltpu.make_async_copy(rows.at[b], acc_sh.at[didx.at[0]],
                                  ssem.at[b]).wait()

        for b in range(NBUF):
            g_issue(b, b)

        for b in range(NB2):
            g_wait(b)
            s_issue(b, b)
            nx = b + NBUF
            if nx < NB2:
                g_issue(nx, nx)
            else:
                sn = nx % NB2
                s_wait(sn)
                g_issue(sn, nx)

        @pl.loop(1, NCHUNK // NB2)
        def _(r):
            for b in range(NB2):
                jj = r * NB2 + b
                g_wait(b)
                s_issue(b, jj)
                sn = (b + NBUF) % NB2
                s_wait(sn)
                g_issue(sn, jj + NBUF)

        for b in range(NBUF):
            jj = (NCHUNK // NB2) * NB2 + b
            g_wait(b)
            s_issue(b, jj)

        for b in range(NB2):
            s_wait(b)

        plsc.subcore_barrier()
        _drain_shared(acc_sh, zbuf, out_hbm, core, sid)

    return k(hp, src3, dst3)


def _sc_edge_pre(A3, B3, src3, dst3):
    @functools.partial(
        pl.kernel,
        out_type=jax.ShapeDtypeStruct((E, 16), jnp.float32),
        mesh=_mesh,
        compiler_params=_sc_params,
        scratch_types=[
            pltpu.VMEM((NCHUNK, CHUNK), jnp.int32),
            pltpu.VMEM((NCHUNK, CHUNK), jnp.int32),
            pltpu.VMEM((NBUF, CHUNK, 16), jnp.float32),
            pltpu.VMEM((NBUF, CHUNK, 16), jnp.float32),
            pltpu.VMEM((NBUF, CHUNK, 16), jnp.float32),
            pltpu.SemaphoreType.DMA((NBUF,)),
            pltpu.SemaphoreType.DMA((NBUF,)),
            pltpu.SemaphoreType.DMA((NBUF,)),
        ],
    )
    def k(a_hbm, b_hbm, src_hbm, dst_hbm, out_hbm, sidx, didx, ga, gb, wo,
          gsa, gsb, wsem):
        core = lax.axis_index("c")
        sid = lax.axis_index("s")
        wid = core * NS + sid
        base = wid * EPW

        pltpu.sync_copy(src_hbm.at[wid], sidx)
        pltpu.sync_copy(dst_hbm.at[wid], didx)

        def issue(b, jj):
            pltpu.async_copy(a_hbm.at[sidx.at[jj]], ga.at[b], gsa.at[b])
            pltpu.async_copy(b_hbm.at[didx.at[jj]], gb.at[b], gsb.at[b])

        def out_slice(jj):
            return out_hbm.at[pl.ds(base + jj * CHUNK, CHUNK)]

        def process(jj, b, first):
            pltpu.make_async_copy(a_hbm.at[sidx.at[0]], ga.at[b],
                                  gsa.at[b]).wait()
            pltpu.make_async_copy(b_hbm.at[didx.at[0]], gb.at[b],
                                  gsb.at[b]).wait()
            if not first:
                pltpu.make_async_copy(wo.at[b], out_slice(jj),
                                      wsem.at[b]).wait()

            @pl.loop(0, CHUNK)
            def _(c):
                wo.at[b][c] = ga.at[b][c] + gb.at[b][c]

            pltpu.async_copy(wo.at[b], out_slice(jj), wsem.at[b])

        for b in range(NBUF):
            issue(b, b)
        for b in range(NBUF):
            process(b, b, True)
            issue(b, b + NBUF)

        @pl.loop(1, NROUND - 1)
        def _(r):
            for b in range(NBUF):
                jj = r * NBUF + b
                process(jj, b, False)
                issue(b, jj + NBUF)

        for b in range(NBUF):
            jj = (NROUND - 1) * NBUF + b
            process(jj, b, False)
        for b in range(NBUF):
            pltpu.make_async_copy(wo.at[b], out_slice(0), wsem.at[b]).wait()

    return k(A3, B3, src3, dst3)


def _tc_pre(deg_parts, x, W1, b1):
    def body(dp_ref, x_ref, w_ref, dinv_ref, hp_ref):
        indeg = dp_ref[0, :PN, :] + dp_ref[1, :PN, :]
        dinv = lax.rsqrt(indeg + 1.0)
        dinv_ref[...] = dinv
        hw = jnp.dot(x_ref[...], w_ref[...], preferred_element_type=jnp.float32)
        hp_ref[...] = dinv * hw

    return pl.pallas_call(
        body,
        out_shape=(
            jax.ShapeDtypeStruct((PN, 128), jnp.float32),
            jax.ShapeDtypeStruct((PN, 128), jnp.float32),
        ),
    )(deg_parts, x, W1)


def _tc_post(parts, hp, dinv, b128, Wbd, res=None):
    args = [parts, hp, dinv, b128, Wbd] + ([res] if res is not None else [])

    def body(p_ref, hp_ref, dinv_ref, b_ref, w_ref, *rest):
        (res_ref, h_ref, hpn_ref) = rest if len(rest) == 3 else \
            (None,) + rest
        acc = p_ref[0, :PN, :] + p_ref[1, :PN, :] + hp_ref[...]
        out = dinv_ref[...] * acc + b_ref[...]
        h = jnp.maximum(out, 0.0)
        if res_ref is not None:
            h = h + res_ref[...]
        h_ref[...] = h
        hw = jnp.dot(h, w_ref[...], preferred_element_type=jnp.float32)
        hpn_ref[...] = dinv_ref[...] * hw

    return pl.pallas_call(
        body,
        out_shape=(
            jax.ShapeDtypeStruct((PN, 128), jnp.float32),
            jax.ShapeDtypeStruct((PN, 128), jnp.float32),
        ),
    )(*args)


def _tc_post3(parts, hp, dinv, b128, Wabd, Wbbd, bf1_128):
    def body(p_ref, hp_ref, dinv_ref, b_ref, wa_ref, wb_ref, bf1_ref,
             a_ref, bo_ref):
        acc = p_ref[0, :PN, :] + p_ref[1, :PN, :] + hp_ref[...]
        h3 = dinv_ref[...] * acc + b_ref[...]
        a_ref[...] = jnp.dot(h3, wa_ref[...],
                             preferred_element_type=jnp.float32) + bf1_ref[...]
        bo_ref[...] = jnp.dot(h3, wb_ref[...],
                              preferred_element_type=jnp.float32)

    return pl.pallas_call(
        body,
        out_shape=(
            jax.ShapeDtypeStruct((PN, 128), jnp.float32),
            jax.ShapeDtypeStruct((PN, 128), jnp.float32),
        ),
    )(parts, hp, dinv, b128, Wabd, Wbbd, bf1_128)


EDGE_BLOCK = 4000


def _group_perm(k):
    j = np.arange(128)
    src = (j // 16) * 16 + ((j % 16 + k) % 16)
    p = np.zeros((128, 128), np.float32)
    p[src, j] = 1.0
    return jnp.asarray(p)


def _tc_final(pre_p, Wf2bd, bf2_128, bdones, perms):
    def body(pre_ref, w_ref, b_ref, ones_ref, p1, p2, p4, p8, out_ref):
        ef = jnp.maximum(pre_ref[...], 0.0)
        logits = jnp.dot(ef, w_ref[...], preferred_element_type=jnp.float32)
        logits = logits + b_ref[...]
        m = logits
        for p_ref in (p1, p2, p4, p8):
            m = jnp.maximum(m, jnp.dot(m, p_ref[...],
                                       preferred_element_type=jnp.float32))
        s = logits - m
        se = jnp.dot(jnp.exp(s), ones_ref[...],
                     preferred_element_type=jnp.float32)
        out_ref[...] = s - jnp.log(se)

    full = lambda i: (0, 0)
    return pl.pallas_call(
        body,
        grid=(PE // EDGE_BLOCK,),
        in_specs=[pl.BlockSpec((EDGE_BLOCK, 128), lambda i: (i, 0))] +
                 [pl.BlockSpec((128, 128), full)] +
                 [pl.BlockSpec((1, 128), full)] +
                 [pl.BlockSpec((128, 128), full)] * 5,
        out_specs=pl.BlockSpec((EDGE_BLOCK, 128), lambda i: (i, 0)),
        out_shape=jax.ShapeDtypeStruct((PE, 128), jnp.float32),
    )(pre_p, Wf2bd, bf2_128, bdones, *perms)


def _bd(W):
    return jnp.kron(jnp.eye(8, dtype=W.dtype), W)


def kernel(x, edge_index, W1, b1, W2, b2, W3, b3, Wf1, bf1, Wf2, bf2):
    src3 = edge_index[0].astype(jnp.int32).reshape(NW, NCHUNK, CHUNK)
    dst3 = edge_index[1].astype(jnp.int32).reshape(NW, NCHUNK, CHUNK)

    x_r = x.reshape(PN, 8 * F)
    W1bd = _bd(W1)
    W2bd, W3bd = _bd(W2), _bd(W3)
    Wabd, Wbbd = _bd(Wf1[:16]), _bd(Wf1[16:])
    Wf2bd = _bd(Wf2)
    t8 = lambda b: jnp.tile(b, 8).reshape(1, 128)

    deg_parts = _sc_degree(dst3).reshape(NC, PP, 128)
    dinv, hp1 = _tc_pre(deg_parts, x_r, W1bd, b1)

    p1 = _sc_conv(hp1.reshape(N, 16), src3, dst3).reshape(NC, PP, 128)
    h1, hp2 = _tc_post(p1, hp1, dinv, t8(b1), W2bd)

    p2 = _sc_conv(hp2.reshape(N, 16), src3, dst3).reshape(NC, PP, 128)
    h2, hp3 = _tc_post(p2, hp2, dinv, t8(b2), W3bd, res=h1)

    p3 = _sc_conv(hp3.reshape(N, 16), src3, dst3).reshape(NC, PP, 128)
    A3, B3 = _tc_post3(p3, hp3, dinv, t8(b3), Wabd, Wbbd, t8(bf1))

    pre = _sc_edge_pre(A3.reshape(N, 16), B3.reshape(N, 16), src3, dst3)
    bdones = jnp.asarray(np.kron(np.eye(8, dtype=np.float32),
                                 np.ones((16, 16), np.float32)))
    perms = [_group_perm(k) for k in (1, 2, 4, 8)]
    out_p = _tc_final(pre.reshape(PE, 128), Wf2bd, t8(bf2), bdones, perms)
    return out_p.reshape(E, 16)


# speedup vs baseline: 1.0825x; 1.0199x over previous
import functools

import numpy as np

import jax
import jax.numpy as jnp
from jax import lax
from jax.experimental import pallas as pl
from jax.experimental.pallas import tpu as pltpu
from jax.experimental.pallas import tpu_sc as plsc

N = 10000
E = 320000
F = 128
H = 16

NC, NS = 2, 16
NW = NC * NS
EPW = E // NW
CHUNK = 80
NCHUNK = EPW // CHUNK
RPW = 632
NPAD = NS * RPW
PN = N // 8
PP = NPAD // 8
PE = E // 8

_mesh = plsc.VectorSubcoreMesh(core_axis_name="c", subcore_axis_name="s")
_sc_params = pltpu.CompilerParams(use_tc_tiling_on_sc=False)


def _zero_shared(acc_sh, zbuf, sid):
    zrow = jnp.zeros((16,), jnp.float32)

    @pl.loop(0, RPW)
    def _(i):
        zbuf[i] = zrow

    pltpu.sync_copy(zbuf, acc_sh.at[pl.ds(sid * RPW, RPW)])


def _drain_shared(acc_sh, zbuf, out_hbm, core, sid):
    sl = pl.ds(sid * RPW, RPW)
    pltpu.sync_copy(acc_sh.at[sl], zbuf)
    pltpu.sync_copy(zbuf, out_hbm.at[core, sl])


def _sc_degree(dst3):
    @functools.partial(
        pl.kernel,
        out_type=jax.ShapeDtypeStruct((NC, NPAD, 16), jnp.float32),
        mesh=_mesh,
        compiler_params=_sc_params,
        scratch_types=[
            pltpu.VMEM((RPW, 16), jnp.float32),
            pltpu.VMEM((NCHUNK, CHUNK), jnp.int32),
            pltpu.VMEM((CHUNK, 16), jnp.float32),
            pltpu.VMEM_SHARED((NPAD, 16), jnp.float32),
            pltpu.SemaphoreType.DMA((4,)),
        ],
    )
    def k(dst_hbm, out_hbm, zbuf, didx, ones_v, acc_sh, ssem):
        core = lax.axis_index("c")
        sid = lax.axis_index("s")
        wid = core * NS + sid

        _zero_shared(acc_sh, zbuf, sid)

        one = jnp.ones((16,), jnp.float32)

        @pl.loop(0, CHUNK)
        def _(i):
            ones_v[i] = one

        pltpu.sync_copy(dst_hbm.at[wid], didx)
        plsc.subcore_barrier()

        def s_wait(b):
            pltpu.make_async_copy(ones_v, acc_sh.at[didx.at[0]],
                                  ssem.at[b]).wait()

        @pl.loop(0, NCHUNK)
        def _(j):
            b = lax.rem(j, 4)

            @pl.when(j >= 4)
            def _():
                s_wait(b)

            pltpu.async_copy(ones_v, acc_sh.at[didx.at[j]], ssem.at[b],
                             add=True)

        for b in range(4):
            s_wait(b)

        plsc.subcore_barrier()
        _drain_shared(acc_sh, zbuf, out_hbm, core, sid)

    return k(dst3)


NBUF = 5
NROUND = NCHUNK // NBUF


def _sc_conv(hp, src3, dst3):
    NB2 = 2 * NBUF

    @functools.partial(
        pl.kernel,
        out_type=jax.ShapeDtypeStruct((NC, NPAD, 16), jnp.float32),
        mesh=_mesh,
        compiler_params=_sc_params,
        scratch_types=[
            pltpu.VMEM((RPW, 16), jnp.float32),
            pltpu.VMEM((NCHUNK, CHUNK), jnp.int32),
            pltpu.VMEM((NCHUNK, CHUNK), jnp.int32),
            pltpu.VMEM((2 * NBUF, CHUNK, 16), jnp.float32),
            pltpu.VMEM_SHARED((NPAD, 16), jnp.float32),
            pltpu.SemaphoreType.DMA((2 * NBUF,)),
            pltpu.SemaphoreType.DMA((2 * NBUF,)),
        ],
    )
    def k(hp_hbm, src_hbm, dst_hbm, out_hbm, zbuf, sidx, didx, rows, acc_sh,
          gsem, ssem):
        core = lax.axis_index("c")
        sid = lax.axis_index("s")
        wid = core * NS + sid

        _zero_shared(acc_sh, zbuf, sid)
        pltpu.sync_copy(src_hbm.at[wid], sidx)
        pltpu.sync_copy(dst_hbm.at[wid], didx)
        plsc.subcore_barrier()

        def g_issue(b, jj):
            pltpu.async_copy(hp_hbm.at[sidx.at[jj]], rows.at[b], gsem.at[b])

        def g_wait(b):
            pltpu.make_async_copy(hp_hbm.at[sidx.at[0]], rows.at[b],
                                  gsem.at[b]).wait()

        def s_issue(b, jj):
            pltpu.async_copy(rows.at[b], acc_sh.at[didx.at[jj]], ssem.at[b],
                             add=True)

        def s_wait(b):
            pltpu.make_async_copy(rows.at[b], acc_sh.at[didx.at[0]],
                                  ssem.at[b]).wait()

        for b in range(NBUF):
            g_issue(b, b)

        for b in range(NB2):
            g_wait(b)
            s_issue(b, b)
            nx = b + NBUF
            if nx < NB2:
                g_issue(nx, nx)
            else:
                sn = nx % NB2
                s_wait(sn)
                g_issue(sn, nx)

        @pl.loop(1, NCHUNK // NB2)
        def _(r):
            for b in range(NB2):
                jj = r * NB2 + b
                g_wait(b)
                s_issue(b, jj)
                sn = (b + NBUF) % NB2
                s_wait(sn)
                g_issue(sn, jj + NBUF)

        for b in range(NBUF):
            jj = (NCHUNK // NB2) * NB2 + b
            g_wait(b)
            s_issue(b, jj)

        for b in range(NB2):
            s_wait(b)

        plsc.subcore_barrier()
        _drain_shared(acc_sh, zbuf, out_hbm, core, sid)

    return k(hp, src3, dst3)


def _sc_edge_pre(A3, B3, src3, dst3):
    @functools.partial(
        pl.kernel,
        out_type=jax.ShapeDtypeStruct((E, 16), jnp.float32),
        mesh=_mesh,
        compiler_params=_sc_params,
        scratch_types=[
            pltpu.VMEM((NCHUNK, CHUNK), jnp.int32),
            pltpu.VMEM((NCHUNK, CHUNK), jnp.int32),
            pltpu.VMEM((NBUF, CHUNK, 16), jnp.float32),
            pltpu.VMEM((NBUF, CHUNK, 16), jnp.float32),
            pltpu.VMEM((NBUF, CHUNK, 16), jnp.float32),
            pltpu.SemaphoreType.DMA((NBUF,)),
            pltpu.SemaphoreType.DMA((NBUF,)),
            pltpu.SemaphoreType.DMA((NBUF,)),
        ],
    )
    def k(a_hbm, b_hbm, src_hbm, dst_hbm, out_hbm, sidx, didx, ga, gb, wo,
          gsa, gsb, wsem):
        core = lax.axis_index("c")
        sid = lax.axis_index("s")
        wid = core * NS + sid
        base = wid * EPW

        pltpu.sync_copy(src_hbm.at[wid], sidx)
        pltpu.sync_copy(dst_hbm.at[wid], didx)

        def issue(b, jj):
            pltpu.async_copy(a_hbm.at[sidx.at[jj]], ga.at[b], gsa.at[b])
            pltpu.async_copy(b_hbm.at[didx.at[jj]], gb.at[b], gsb.at[b])

        def out_slice(jj):
            return out_hbm.at[pl.ds(base + jj * CHUNK, CHUNK)]

        def process(jj, b, first):
            pltpu.make_async_copy(a_hbm.at[sidx.at[0]], ga.at[b],
                                  gsa.at[b]).wait()
            pltpu.make_async_copy(b_hbm.at[didx.at[0]], gb.at[b],
                                  gsb.at[b]).wait()
            if not first:
                pltpu.make_async_copy(wo.at[b], out_slice(jj),
                                      wsem.at[b]).wait()

            @pl.loop(0, CHUNK)
            def _(c):
                wo.at[b][c] = ga.at[b][c] + gb.at[b][c]

            pltpu.async_copy(wo.at[b], out_slice(jj), wsem.at[b])

        for b in range(NBUF):
            issue(b, b)
        for b in range(NBUF):
            process(b, b, True)
            issue(b, b + NBUF)

        @pl.loop(1, NROUND - 1)
        def _(r):
            for b in range(NBUF):
                jj = r * NBUF + b
                process(jj, b, False)
                issue(b, jj + NBUF)

        for b in range(NBUF):
            jj = (NROUND - 1) * NBUF + b
            process(jj, b, False)
        for b in range(NBUF):
            pltpu.make_async_copy(wo.at[b], out_slice(0), wsem.at[b]).wait()

    return k(A3, B3, src3, dst3)


def _tc_pre(deg_parts, x, W1, b1):
    def body(dp_ref, x_ref, w_ref, dinv_ref, hp_ref):
        indeg = dp_ref[0, :PN, :] + dp_ref[1, :PN, :]
        dinv = lax.rsqrt(indeg + 1.0)
        dinv_ref[...] = dinv
        hw = jnp.dot(x_ref[...], w_ref[...], preferred_element_type=jnp.float32)
        hp_ref[...] = dinv * hw

    return pl.pallas_call(
        body,
        out_shape=(
            jax.ShapeDtypeStruct((PN, 128), jnp.float32),
            jax.ShapeDtypeStruct((PN, 128), jnp.float32),
        ),
    )(deg_parts, x, W1)


def _tc_post(parts, hp, dinv, b128, Wbd, res=None):
    args = [parts, hp, dinv, b128, Wbd] + ([res] if res is not None else [])

    def body(p_ref, hp_ref, dinv_ref, b_ref, w_ref, *rest):
        (res_ref, h_ref, hpn_ref) = rest if len(rest) == 3 else \
            (None,) + rest
        acc = p_ref[0, :PN, :] + p_ref[1, :PN, :] + hp_ref[...]
        out = dinv_ref[...] * acc + b_ref[...]
        h = jnp.maximum(out, 0.0)
        if res_ref is not None:
            h = h + res_ref[...]
        h_ref[...] = h
        hw = jnp.dot(h, w_ref[...], preferred_element_type=jnp.float32)
        hpn_ref[...] = dinv_ref[...] * hw

    return pl.pallas_call(
        body,
        out_shape=(
            jax.ShapeDtypeStruct((PN, 128), jnp.float32),
            jax.ShapeDtypeStruct((PN, 128), jnp.float32),
        ),
    )(*args)


def _tc_post3(parts, hp, dinv, b128, Wabd, Wbbd, bf1_128):
    def body(p_ref, hp_ref, dinv_ref, b_ref, wa_ref, wb_ref, bf1_ref,
             a_ref, bo_ref):
        acc = p_ref[0, :PN, :] + p_ref[1, :PN, :] + hp_ref[...]
        h3 = dinv_ref[...] * acc + b_ref[...]
        a_ref[...] = jnp.dot(h3, wa_ref[...],
                             preferred_element_type=jnp.float32) + bf1_ref[...]
        bo_ref[...] = jnp.dot(h3, wb_ref[...],
                              preferred_element_type=jnp.float32)

    return pl.pallas_call(
        body,
        out_shape=(
            jax.ShapeDtypeStruct((PN, 128), jnp.float32),
            jax.ShapeDtypeStruct((PN, 128), jnp.float32),
        ),
    )(parts, hp, dinv, b128, Wabd, Wbbd, bf1_128)


EDGE_BLOCK = 4000


def _group_perm(k):
    j = np.arange(128)
    src = (j // 16) * 16 + ((j % 16 + k) % 16)
    p = np.zeros((128, 128), np.float32)
    p[src, j] = 1.0
    return jnp.asarray(p)


def _tc_final(pre_p, Wf2bd, bf2_128, bdones, perms):
    def body(pre_ref, w_ref, b_ref, ones_ref, p1, p2, p4, p8, out_ref):
        ef = jnp.maximum(pre_ref[...], 0.0)
        logits = jnp.dot(ef, w_ref[...], preferred_element_type=jnp.float32)
        logits = logits + b_ref[...]
        m = logits
        for p_ref in (p1, p2, p4, p8):
            m = jnp.maximum(m, jnp.dot(m, p_ref[...],
                                       preferred_element_type=jnp.float32))
        s = logits - m
        se = jnp.dot(jnp.exp(s), ones_ref[...],
                     preferred_element_type=jnp.float32)
        out_ref[...] = s - jnp.log(se)

    full = lambda i: (0, 0)
    return pl.pallas_call(
        body,
        grid=(PE // EDGE_BLOCK,),
        in_specs=[pl.BlockSpec((EDGE_BLOCK, 128), lambda i: (i, 0))] +
                 [pl.BlockSpec((128, 128), full)] +
                 [pl.BlockSpec((1, 128), full)] +
                 [pl.BlockSpec((128, 128), full)] * 5,
        out_specs=pl.BlockSpec((EDGE_BLOCK, 128), lambda i: (i, 0)),
        out_shape=jax.ShapeDtypeStruct((PE, 128), jnp.float32),
    )(pre_p, Wf2bd, bf2_128, bdones, *perms)


def _bd(W):
    return jnp.kron(jnp.eye(8, dtype=W.dtype), W)


def kernel(x, edge_index, W1, b1, W2, b2, W3, b3, Wf1, bf1, Wf2, bf2):
    src3 = edge_index[0].astype(jnp.int32).reshape(NW, NCHUNK, CHUNK)
    dst3 = edge_index[1].astype(jnp.int32).reshape(NW, NCHUNK, CHUNK)

    x_r = x.reshape(PN, 8 * F)
    W1bd = _bd(W1)
    W2bd, W3bd = _bd(W2), _bd(W3)
    Wabd, Wbbd = _bd(Wf1[:16]), _bd(Wf1[16:])
    Wf2bd = _bd(Wf2)
    t8 = lambda b: jnp.tile(b, 8).reshape(1, 128)

    deg_parts = _sc_degree(dst3).reshape(NC, PP, 128)
    dinv, hp1 = _tc_pre(deg_parts, x_r, W1bd, b1)

    p1 = _sc_conv(hp1.reshape(N, 16), src3, dst3).reshape(NC, PP, 128)
    h1, hp2 = _tc_post(p1, hp1, dinv, t8(b1), W2bd)

    p2 = _sc_conv(hp2.reshape(N, 16), src3, dst3).reshape(NC, PP, 128)
    h2, hp3 = _tc_post(p2, hp2, dinv, t8(b2), W3bd, res=h1)

    p3 = _sc_conv(hp3.reshape(N, 16), src3, dst3).reshape(NC, PP, 128)
    A3, B3 = _tc_post3(p3, hp3, dinv, t8(b3), Wabd, Wbbd, t8(bf1))

    pre = _sc_edge_pre(A3.reshape(N, 16), B3.reshape(N, 16), src3, dst3)
    bdones = jnp.asarray(np.kron(np.eye(8, dtype=np.float32),
                                 np.ones((16, 16), np.float32)))
    perms = [_group_perm(k) for k in (1, 2, 4, 8)]
    out_p = _tc_final(pre.reshape(PE, 128), Wf2bd, t8(bf2), bdones, perms)
    return out_p.reshape(E, 16)
